# baseline (speedup 1.0000x reference)
import functools

import jax
import jax.numpy as jnp
from jax import lax
from jax.experimental import pallas as pl
from jax.experimental.pallas import tpu as pltpu
from jax.experimental.pallas import tpu_sc as plsc

B = 1024
N_FEAT_ROWS = 10000
K = 32
D = 512
JC = 33
JG = 40
NC = 2
NS = 16
NW = NC * NS
L = 16
NBUF = 4

BSC = B
BPW = BSC // NW
JP = 72
NP = BPW // 2


def _weights_body(nb_ref, nd_ref, w_ref):
    nb = nb_ref[...]
    nd = nd_ref[...]
    self_match = (nb == nd).astype(jnp.float32)
    cnt = self_match
    for k in range(K):
        cnt = cnt + (nb == nb[:, k:k + 1]).astype(jnp.float32)
    inv_nb = 1.0 / cnt
    cnt_self = 1.0 + jnp.sum(self_match, axis=1, keepdims=True)
    inv_self = 1.0 / cnt_self
    c = jnp.sum(inv_nb, axis=1, keepdims=True) + inv_self
    w_ref[...] = jnp.concatenate([inv_nb / c, inv_self / c], axis=1)


_weights = pl.pallas_call(
    _weights_body,
    out_shape=jax.ShapeDtypeStruct((B, JC), jnp.float32),
)


@functools.partial(
    pl.kernel,
    out_type=jax.ShapeDtypeStruct((BSC, D), jnp.float32),
    mesh=plsc.VectorSubcoreMesh(core_axis_name="c", subcore_axis_name="s"),
    scratch_types=[
        pltpu.VMEM((NP * JP,), jnp.int32),
        pltpu.VMEM((BPW * JC * L,), jnp.float32),
        pltpu.VMEM((2, JP, D), jnp.float32),
        pltpu.VMEM((BPW, D), jnp.float32),
        pltpu.SemaphoreType.DMA,
        pltpu.SemaphoreType.DMA,
    ],
)
def _sc_aggregate(feat_hbm, ids_hbm, w_hbm, out_hbm,
                  ids_v, w_v, rows_v, obuf_v, sem0, sem1):
    wid = lax.axis_index("s") * NC + lax.axis_index("c")
    base = wid * BPW
    pltpu.sync_copy(ids_hbm.at[pl.ds(wid * NP * JP, NP * JP)], ids_v)
    pltpu.sync_copy(w_hbm.at[pl.ds(base * JC * L, BPW * JC * L)], w_v)

    def gather(p, buf, sem):
        pltpu.async_copy(
            feat_hbm.at[ids_v.at[pl.ds(p * JP, JP)]],
            rows_v.at[buf], sem)

    def gather_wait(buf, sem):
        pltpu.make_async_copy(
            feat_hbm.at[ids_v.at[pl.ds(0, JP)]], rows_v.at[buf], sem
        ).wait()

    def compute(r, buf, half):
        wvs = [w_v[pl.ds((r * JC + j) * L, L)] for j in range(JC)]
        h = half * JC

        def cc_body(cc, c3):
            off = cc * L
            acc0 = wvs[0] * rows_v[buf, h, pl.ds(off, L)]
            acc1 = wvs[1] * rows_v[buf, h + 1, pl.ds(off, L)]
            for j in range(2, JC, 2):
                acc0 = acc0 + wvs[j] * rows_v[buf, h + j, pl.ds(off, L)]
            for j in range(3, JC, 2):
                acc1 = acc1 + wvs[j] * rows_v[buf, h + j, pl.ds(off, L)]
            obuf_v[r, pl.ds(off, L)] = acc0 + acc1
            return c3

        lax.fori_loop(0, D // L, cc_body, 0)

    gather(0, 0, sem0)

    def quad_body(q, carry):
        p0 = 2 * q
        p1 = p0 + 1
        gather(p1, 1, sem1)
        gather_wait(0, sem0)
        compute(2 * p0, 0, 0)
        compute(2 * p0 + 1, 0, 1)

        @pl.when(q < NP // 2 - 1)
        def _():
            gather(p0 + 2, 0, sem0)

        gather_wait(1, sem1)
        compute(2 * p1, 1, 0)
        compute(2 * p1 + 1, 1, 1)
        return carry

    lax.fori_loop(0, NP // 2, quad_body, 0)
    pltpu.sync_copy(obuf_v, out_hbm.at[pl.ds(base, BPW)])


def kernel(raw_features, nodes, neighbors):
    nb = neighbors.astype(jnp.int32)
    nd = nodes.astype(jnp.int32).reshape(B, 1)
    w = _weights(nb, nd)
    ids33 = jnp.concatenate([nb, nd], axis=1)

    ids66 = ids33.reshape(B // 2, 2 * JC)
    npad = JP - 2 * JC
    pads = (jnp.arange(B // 2, dtype=jnp.int32)[:, None] * npad
            + jnp.arange(npad, dtype=jnp.int32)[None, :]) % N_FEAT_ROWS
    ids_sc = jnp.concatenate([ids66, pads], axis=1).reshape(B // 2 * JP)
    w_sc = jnp.broadcast_to(
        w[:, :, None], (BSC, JC, L)).reshape(BSC * JC * L)

    return _sc_aggregate(raw_features, ids_sc, w_sc)


# speedup vs baseline: 1.0017x; 1.0017x over previous
import functools

import jax
import jax.numpy as jnp
from jax import lax
from jax.experimental import pallas as pl
from jax.experimental.pallas import tpu as pltpu
from jax.experimental.pallas import tpu_sc as plsc

B = 1024
N_FEAT_ROWS = 10000
K = 32
D = 512
JC = 33
JG = 40
NC = 2
NS = 16
NW = NC * NS
L = 16
NBUF = 4

BSC = B
BPW = BSC // NW
JP = 72
NP = BPW // 2


def _weights_body(nb_ref, nd_ref, w_ref):
    nb = nb_ref[...]
    nd = nd_ref[...]
    self_match = (nb == nd).astype(jnp.float32)
    cnt = self_match
    for k in range(K):
        cnt = cnt + (nb == nb[:, k:k + 1]).astype(jnp.float32)
    inv_nb = 1.0 / cnt
    cnt_self = 1.0 + jnp.sum(self_match, axis=1, keepdims=True)
    inv_self = 1.0 / cnt_self
    c = jnp.sum(inv_nb, axis=1, keepdims=True) + inv_self
    w_ref[...] = jnp.concatenate([inv_nb / c, inv_self / c], axis=1)


_weights = pl.pallas_call(
    _weights_body,
    out_shape=jax.ShapeDtypeStruct((B, JC), jnp.float32),
)


@functools.partial(
    pl.kernel,
    out_type=jax.ShapeDtypeStruct((BSC, D), jnp.float32),
    mesh=plsc.VectorSubcoreMesh(core_axis_name="c", subcore_axis_name="s"),
    scratch_types=[
        pltpu.VMEM((NP * JP,), jnp.int32),
        pltpu.VMEM((BPW * JC * L,), jnp.float32),
        pltpu.VMEM((2, JP, D), jnp.float32),
        pltpu.VMEM((BPW, D), jnp.float32),
        pltpu.SemaphoreType.DMA,
        pltpu.SemaphoreType.DMA,
    ],
)
def _sc_aggregate(feat_hbm, ids_hbm, w_hbm, out_hbm,
                  ids_v, w_v, rows_v, obuf_v, sem0, sem1):
    wid = lax.axis_index("s") * NC + lax.axis_index("c")
    base = wid * BPW
    pltpu.sync_copy(ids_hbm.at[pl.ds(wid * NP * JP, NP * JP)], ids_v)
    pltpu.sync_copy(w_hbm.at[pl.ds(base * JC * L, BPW * JC * L)], w_v)

    def gather(p, buf, sem):
        pltpu.async_copy(
            feat_hbm.at[ids_v.at[pl.ds(p * JP, 40)]],
            rows_v.at[buf, pl.ds(0, 40)], sem)
        pltpu.async_copy(
            feat_hbm.at[ids_v.at[pl.ds(p * JP + 40, JP - 40)]],
            rows_v.at[buf, pl.ds(40, JP - 40)], sem)

    def gather_wait(buf, sem):
        pltpu.make_async_copy(
            feat_hbm.at[ids_v.at[pl.ds(0, JP)]], rows_v.at[buf], sem
        ).wait()

    def compute(r, buf, half):
        wvs = [w_v[pl.ds((r * JC + j) * L, L)] for j in range(JC)]
        h = half * JC

        def cc_body(cc, c3):
            off = cc * L
            acc0 = wvs[0] * rows_v[buf, h, pl.ds(off, L)]
            acc1 = wvs[1] * rows_v[buf, h + 1, pl.ds(off, L)]
            for j in range(2, JC, 2):
                acc0 = acc0 + wvs[j] * rows_v[buf, h + j, pl.ds(off, L)]
            for j in range(3, JC, 2):
                acc1 = acc1 + wvs[j] * rows_v[buf, h + j, pl.ds(off, L)]
            obuf_v[r, pl.ds(off, L)] = acc0 + acc1
            return c3

        lax.fori_loop(0, D // L, cc_body, 0)

    gather(0, 0, sem0)

    def quad_body(q, carry):
        p0 = 2 * q
        p1 = p0 + 1
        gather(p1, 1, sem1)
        gather_wait(0, sem0)
        compute(2 * p0, 0, 0)
        compute(2 * p0 + 1, 0, 1)

        @pl.when(q < NP // 2 - 1)
        def _():
            gather(p0 + 2, 0, sem0)

        gather_wait(1, sem1)
        compute(2 * p1, 1, 0)
        compute(2 * p1 + 1, 1, 1)
        return carry

    lax.fori_loop(0, NP // 2, quad_body, 0)
    pltpu.sync_copy(obuf_v, out_hbm.at[pl.ds(base, BPW)])


def kernel(raw_features, nodes, neighbors):
    nb = neighbors.astype(jnp.int32)
    nd = nodes.astype(jnp.int32).reshape(B, 1)
    w = _weights(nb, nd)
    ids33 = jnp.concatenate([nb, nd], axis=1)

    ids66 = ids33.reshape(B // 2, 2 * JC)
    npad = JP - 2 * JC
    pads = (jnp.arange(B // 2, dtype=jnp.int32)[:, None] * npad
            + jnp.arange(npad, dtype=jnp.int32)[None, :]) % N_FEAT_ROWS
    ids_sc = jnp.concatenate([ids66, pads], axis=1).reshape(B // 2 * JP)
    w_sc = jnp.broadcast_to(
        w[:, :, None], (BSC, JC, L)).reshape(BSC * JC * L)

    return _sc_aggregate(raw_features, ids_sc, w_sc)


# speedup vs baseline: 1.0542x; 1.0524x over previous
import functools

import jax
import jax.numpy as jnp
from jax import lax
from jax.experimental import pallas as pl
from jax.experimental.pallas import tpu as pltpu
from jax.experimental.pallas import tpu_sc as plsc

B = 1024
N_FEAT_ROWS = 10000
K = 32
D = 512
JC = 33
JG = 40
NC = 2
NS = 16
NW = NC * NS
L = 16
NBUF = 4

BSC = B
BPW = BSC // NW
JP = 72
NP = BPW // 2


def _weights_body(nb_ref, nd_ref, w_ref):
    nb = nb_ref[...]
    nd = nd_ref[...]
    self_match = (nb == nd).astype(jnp.float32)
    cnt = self_match
    for k in range(K):
        cnt = cnt + (nb == nb[:, k:k + 1]).astype(jnp.float32)
    inv_nb = 1.0 / cnt
    cnt_self = 1.0 + jnp.sum(self_match, axis=1, keepdims=True)
    inv_self = 1.0 / cnt_self
    c = jnp.sum(inv_nb, axis=1, keepdims=True) + inv_self
    w_ref[...] = jnp.concatenate([inv_nb / c, inv_self / c], axis=1)


_weights = pl.pallas_call(
    _weights_body,
    out_shape=jax.ShapeDtypeStruct((B, JC), jnp.float32),
)


@functools.partial(
    pl.kernel,
    out_type=jax.ShapeDtypeStruct((BSC, D), jnp.float32),
    mesh=plsc.VectorSubcoreMesh(core_axis_name="c", subcore_axis_name="s"),
    scratch_types=[
        pltpu.VMEM((BPW * JG,), jnp.int32),
        pltpu.VMEM((BPW * JC * L,), jnp.float32),
        pltpu.VMEM((NBUF, JG, D), jnp.float32),
        pltpu.VMEM((BPW, D), jnp.float32),
        pltpu.SemaphoreType.DMA,
        pltpu.SemaphoreType.DMA,
        pltpu.SemaphoreType.DMA,
        pltpu.SemaphoreType.DMA,
    ],
)
def _sc_aggregate(feat_hbm, ids_hbm, w_hbm, out_hbm,
                  ids_v, w_v, rows_v, obuf_v, sem0, sem1, sem2, sem3):
    sems = [sem0, sem1, sem2, sem3]
    wid = lax.axis_index("s") * NC + lax.axis_index("c")
    base = wid * BPW
    pltpu.sync_copy(ids_hbm.at[pl.ds(base * JG, BPW * JG)], ids_v)
    pltpu.sync_copy(w_hbm.at[pl.ds(base * JC * L, BPW * JC * L)], w_v)

    def gather(r, buf):
        pltpu.async_copy(
            feat_hbm.at[ids_v.at[pl.ds(r * JG, JG)]],
            rows_v.at[buf], sems[buf])

    def gather_wait(buf):
        pltpu.make_async_copy(
            feat_hbm.at[ids_v.at[pl.ds(0, JG)]], rows_v.at[buf], sems[buf]
        ).wait()

    def compute(r, buf):
        wvs = [w_v[pl.ds((r * JC + j) * L, L)] for j in range(JC)]

        def cc_body(cc, c3):
            off = cc * L
            acc0 = wvs[0] * rows_v[buf, 0, pl.ds(off, L)]
            acc1 = wvs[1] * rows_v[buf, 1, pl.ds(off, L)]
            for j in range(2, JC, 2):
                acc0 = acc0 + wvs[j] * rows_v[buf, j, pl.ds(off, L)]
            for j in range(3, JC, 2):
                acc1 = acc1 + wvs[j] * rows_v[buf, j, pl.ds(off, L)]
            obuf_v[r, pl.ds(off, L)] = acc0 + acc1
            return c3

        lax.fori_loop(0, D // L, cc_body, 0)

    for b in range(NBUF - 1):
        gather(b, b)

    def group_body(q, carry):
        for b in range(NBUF):
            r = NBUF * q + b

            @pl.when(r < BPW - (NBUF - 1))
            def _():
                gather(r + NBUF - 1, (b + NBUF - 1) % NBUF)

            gather_wait(b)
            compute(r, b)
        return carry

    lax.fori_loop(0, BPW // NBUF, group_body, 0)
    pltpu.sync_copy(obuf_v, out_hbm.at[pl.ds(base, BPW)])


def kernel(raw_features, nodes, neighbors):
    nb = neighbors.astype(jnp.int32)
    nd = nodes.astype(jnp.int32).reshape(B, 1)
    w = _weights(nb, nd)
    ids33 = jnp.concatenate([nb, nd], axis=1)

    npad = JG - JC
    pads = (jnp.arange(BSC, dtype=jnp.int32)[:, None] * npad
            + jnp.arange(npad, dtype=jnp.int32)[None, :]) % N_FEAT_ROWS
    ids_sc = jnp.concatenate([ids33, pads], axis=1).reshape(BSC * JG)
    w_sc = jnp.broadcast_to(
        w[:, :, None], (BSC, JC, L)).reshape(BSC * JC * L)

    return _sc_aggregate(raw_features, ids_sc, w_sc)
